# baseline (speedup 1.0000x reference)
import functools

import jax
import jax.numpy as jnp
from jax import lax
from jax.experimental import pallas as pl
from jax.experimental.pallas import tpu as pltpu
from jax.experimental.pallas import tpu_sc as plsc

NBUF = 5
CHUNK = 128
NC = 2
NS = 16
NW = NC * NS
HALF = 32


def _embed_body(cpw, seq_hbm, tl_hbm, tr_hbm, out_hbm, idx_v, bufl, bufr,
                *sems):
    gsems = sems[:NBUF]
    osems = sems[NBUF:]
    c = lax.axis_index("c")
    s = lax.axis_index("s")
    wid = s * NC + c
    row0 = wid * cpw * CHUNK

    pltpu.sync_copy(seq_hbm.at[wid], idx_v)

    def gathers(j, slot):
        lo = pltpu.make_async_copy(
            tl_hbm.at[idx_v.at[j]], bufl.at[slot], gsems[slot])
        hi = pltpu.make_async_copy(
            tr_hbm.at[idx_v.at[j]], bufr.at[slot], gsems[slot])
        return lo, hi

    for slot in range(NBUF):
        for cp in gathers(slot, slot):
            cp.start()

    def one_round(i, refill):
        writes = []
        for slot in range(NBUF):
            j = i * NBUF + slot
            lo, hi = gathers(j, slot)
            lo.wait()
            hi.wait()
            dst = out_hbm.at[pl.ds(row0 + j * CHUNK, CHUNK)]
            writes.append((
                pltpu.async_copy(bufl.at[slot], dst.at[:, pl.ds(0, HALF)],
                                 osems[slot]),
                pltpu.async_copy(bufr.at[slot], dst.at[:, pl.ds(HALF, HALF)],
                                 osems[slot]),
            ))
        for slot in range(NBUF):
            for w in writes[slot]:
                w.wait()
            if refill:
                for cp in gathers((i + 1) * NBUF + slot, slot):
                    cp.start()

    n_rounds = cpw // NBUF
    lax.fori_loop(0, n_rounds - 1, lambda i, _: (one_round(i, True), 0)[1], 0)
    one_round(n_rounds - 1, False)


@jax.jit
def _embed_call(seq3d, table_l, table_r):
    nw, cpw, lanes = seq3d.shape
    grid_kernel = pl.kernel(
        functools.partial(_embed_body, cpw),
        out_type=jax.ShapeDtypeStruct((NW * cpw * CHUNK, 2 * HALF),
                                      jnp.float32),
        mesh=plsc.VectorSubcoreMesh(
            core_axis_name="c", subcore_axis_name="s",
            num_cores=NC, num_subcores=NS,
        ),
        scratch_types=[
            pltpu.VMEM((cpw, CHUNK), jnp.int32),
            pltpu.VMEM((NBUF, CHUNK, HALF), jnp.float32),
            pltpu.VMEM((NBUF, CHUNK, HALF), jnp.float32),
        ] + [pltpu.SemaphoreType.DMA] * (2 * NBUF),
        compiler_params=pltpu.CompilerParams(
            use_tc_tiling_on_sc=False, needs_layout_passes=False
        ),
    )
    return grid_kernel(seq3d, table_l, table_r)


def kernel(seq, table):
    batch, seq_len = seq.shape
    total = batch * seq_len
    assert total % (NW * CHUNK * NBUF) == 0
    seq3d = seq.reshape(NW, total // (NW * CHUNK), CHUNK).astype(jnp.int32)
    out = _embed_call(seq3d, table[:, :HALF], table[:, HALF:])
    return out.reshape(batch, seq_len, table.shape[1])
